# Initial kernel scaffold; baseline (speedup 1.0000x reference)
import jax
import jax.numpy as jnp
from jax.experimental import pallas as pl


def kernel(x, edge_index, W1, b1, W2, b2):
    raise NotImplementedError("write your pallas kernel here")


# speedup vs baseline: 4.7486x; 4.7486x over previous
import jax
import jax.numpy as jnp
from jax import lax
from jax.experimental import pallas as pl
from jax.experimental.pallas import tpu as pltpu
from jax.experimental.pallas import tpu_sc as plsc

N = 10000
E = 160000
D = 256
DH = 128
NS = 16
N_PAD = 10240
STRIPE = N_PAD // NS
ROWS_PAD = 1280
TRIPS = ROWS_PAD // NS
BM = 512
GRID_M = -(-N // BM)

_MESH = plsc.VectorSubcoreMesh(core_axis_name="c", subcore_axis_name="s")


def _agg_body(y0, y1, src2d, dst2d, zeros, out0, out1,
              src_v, dst_v, rows_v, sem, acc):
    cid = lax.axis_index("c")
    sid = lax.axis_index("s")
    stripe = pl.multiple_of(sid * STRIPE, 8)
    rbase = pl.multiple_of(sid * TRIPS, 8)

    pltpu.sync_copy(src2d.at[pl.ds(rbase, TRIPS)], src_v)
    pltpu.sync_copy(dst2d.at[pl.ds(rbase, TRIPS)], dst_v)
    pltpu.sync_copy(zeros, acc.at[pl.ds(stripe, STRIPE)])
    plsc.subcore_barrier()

    def run(y_hbm, out_hbm):
        def body(i, carry):
            pltpu.async_copy(y_hbm.at[src_v.at[i]], rows_v, sem).wait()
            pltpu.sync_copy(rows_v, acc.at[dst_v.at[i]], add=True)
            return carry

        lax.fori_loop(0, TRIPS, body, 0)
        plsc.subcore_barrier()
        pltpu.sync_copy(acc.at[pl.ds(stripe, STRIPE)],
                        out_hbm.at[pl.ds(stripe, STRIPE)])

    @pl.when(cid == 0)
    def _():
        run(y0, out0)

    @pl.when(cid == 1)
    def _():
        run(y1, out1)


_agg_call = pl.kernel(
    _agg_body,
    out_type=[jax.ShapeDtypeStruct((N_PAD, DH), jnp.float32)] * 2,
    mesh=_MESH,
    name="gcn_sc_agg",
    scratch_types=[
        pltpu.VMEM((TRIPS, DH), jnp.int32),
        pltpu.VMEM((TRIPS, DH), jnp.int32),
        pltpu.VMEM((DH, DH), jnp.float32),
        pltpu.SemaphoreType.DMA,
        pltpu.VMEM_SHARED((N_PAD, DH), jnp.float32),
    ],
)


def _mm_body(x_ref, w_ref, o_ref):
    o_ref[...] = jnp.dot(x_ref[...], w_ref[...],
                         preferred_element_type=jnp.float32)


_mm_call = pl.pallas_call(
    _mm_body,
    grid=(GRID_M,),
    in_specs=[
        pl.BlockSpec((BM, D), lambda i: (i, 0)),
        pl.BlockSpec((D, D), lambda i: (0, 0)),
    ],
    out_specs=pl.BlockSpec((BM, D), lambda i: (i, 0)),
    out_shape=jax.ShapeDtypeStruct((N, D), jnp.float32),
)


def _scale_body(xw_ref, deg_ref, y0_ref, y1_ref, dinv_ref):
    dinv1 = lax.rsqrt(deg_ref[...][:, 0:1] + 1.0)
    y = xw_ref[...] * dinv1
    y0_ref[...] = y[:, :DH]
    y1_ref[...] = y[:, DH:]
    dinv_ref[...] = jnp.broadcast_to(dinv1, (BM, 16))


_scale_call = pl.pallas_call(
    _scale_body,
    grid=(GRID_M,),
    in_specs=[
        pl.BlockSpec((BM, D), lambda i: (i, 0)),
        pl.BlockSpec((BM, DH), lambda i: (i, 0)),
    ],
    out_specs=[
        pl.BlockSpec((BM, DH), lambda i: (i, 0)),
        pl.BlockSpec((BM, DH), lambda i: (i, 0)),
        pl.BlockSpec((BM, 16), lambda i: (i, 0)),
    ],
    out_shape=[
        jax.ShapeDtypeStruct((N, DH), jnp.float32),
        jax.ShapeDtypeStruct((N, DH), jnp.float32),
        jax.ShapeDtypeStruct((N, 16), jnp.float32),
    ],
)


def _layer2_body(a0_ref, a1_ref, y0_ref, y1_ref, dinv_ref, b1_ref, w2_ref,
                 o0_ref, o1_ref):
    dinv1 = dinv_ref[...][:, 0:1]
    z = jnp.concatenate([a0_ref[...] + y0_ref[...],
                         a1_ref[...] + y1_ref[...]], axis=1)
    h = jnp.maximum(z * dinv1 + b1_ref[...], 0.0)
    y2 = jnp.dot(h, w2_ref[...], preferred_element_type=jnp.float32) * dinv1
    o0_ref[...] = y2[:, :DH]
    o1_ref[...] = y2[:, DH:]


_layer2_call = pl.pallas_call(
    _layer2_body,
    grid=(GRID_M,),
    in_specs=[
        pl.BlockSpec((BM, DH), lambda i: (i, 0)),
        pl.BlockSpec((BM, DH), lambda i: (i, 0)),
        pl.BlockSpec((BM, DH), lambda i: (i, 0)),
        pl.BlockSpec((BM, DH), lambda i: (i, 0)),
        pl.BlockSpec((BM, 16), lambda i: (i, 0)),
        pl.BlockSpec((1, D), lambda i: (0, 0)),
        pl.BlockSpec((D, D), lambda i: (0, 0)),
    ],
    out_specs=[
        pl.BlockSpec((BM, DH), lambda i: (i, 0)),
        pl.BlockSpec((BM, DH), lambda i: (i, 0)),
    ],
    out_shape=[
        jax.ShapeDtypeStruct((N, DH), jnp.float32),
        jax.ShapeDtypeStruct((N, DH), jnp.float32),
    ],
)


def _final_body(a0_ref, a1_ref, y0_ref, y1_ref, dinv_ref, b2_ref, o_ref):
    dinv1 = dinv_ref[...][:, 0:1]
    z = jnp.concatenate([a0_ref[...] + y0_ref[...],
                         a1_ref[...] + y1_ref[...]], axis=1)
    o_ref[...] = z * dinv1 + b2_ref[...]


_final_call = pl.pallas_call(
    _final_body,
    grid=(GRID_M,),
    in_specs=[
        pl.BlockSpec((BM, DH), lambda i: (i, 0)),
        pl.BlockSpec((BM, DH), lambda i: (i, 0)),
        pl.BlockSpec((BM, DH), lambda i: (i, 0)),
        pl.BlockSpec((BM, DH), lambda i: (i, 0)),
        pl.BlockSpec((BM, 16), lambda i: (i, 0)),
        pl.BlockSpec((1, D), lambda i: (0, 0)),
    ],
    out_specs=pl.BlockSpec((BM, D), lambda i: (i, 0)),
    out_shape=jax.ShapeDtypeStruct((N, D), jnp.float32),
)


@jax.jit
def kernel(x, edge_index, W1, b1, W2, b2):
    src = edge_index[0].astype(jnp.int32)
    dst = edge_index[1].astype(jnp.int32)
    pad = ROWS_PAD * DH - E
    src2d = jnp.concatenate([src, jnp.zeros((pad,), jnp.int32)]).reshape(
        ROWS_PAD, DH)
    dst2d = jnp.concatenate([dst, jnp.full((pad,), N, jnp.int32)]).reshape(
        ROWS_PAD, DH)
    zeros = jnp.zeros((STRIPE, DH), jnp.float32)

    ones = jnp.ones((N, DH), jnp.float32)
    deg128, _ = _agg_call(ones, ones, src2d, dst2d, zeros)
    xw1 = _mm_call(x, W1)
    y0, y1, dinv16 = _scale_call(xw1, deg128)
    a0, a1 = _agg_call(y0, y1, src2d, dst2d, zeros)
    z0, z1 = _layer2_call(a0, a1, y0, y1, dinv16,
                          b1.reshape(1, D), W2)
    c0, c1 = _agg_call(z0, z1, src2d, dst2d, zeros)
    return _final_call(c0, c1, z0, z1, dinv16, b2.reshape(1, D))


# speedup vs baseline: 5.4693x; 1.1518x over previous
import jax
import jax.numpy as jnp
from jax import lax
from jax.experimental import pallas as pl
from jax.experimental.pallas import tpu as pltpu
from jax.experimental.pallas import tpu_sc as plsc

N = 10000
E = 160000
D = 256
DH = 128
NS = 16
N_PAD = 10240
STRIPE = N_PAD // NS
ROWS_PAD = 1280
TRIPS = ROWS_PAD // NS
HALF = TRIPS // 2
BM = 512
GRID_M = -(-N // BM)

_MESH = plsc.VectorSubcoreMesh(core_axis_name="c", subcore_axis_name="s")


def _agg_body(y0, y1, src2d, dst2d, zeros, out0, out1,
              src_v, dst_v, rows_a, rows_b, sem_a, sem_b, acc):
    cid = lax.axis_index("c")
    sid = lax.axis_index("s")
    stripe = pl.multiple_of(sid * STRIPE, 8)
    rbase = pl.multiple_of(sid * TRIPS, 8)

    pltpu.sync_copy(zeros, acc.at[pl.ds(stripe, STRIPE)])
    plsc.subcore_barrier()

    def run(y_hbm, out_hbm):
        bufs = (rows_a, rows_b)
        sems = (sem_a, sem_b)
        for h in range(2):
            hbase = pl.multiple_of(rbase + h * HALF, 8)
            pltpu.sync_copy(src2d.at[pl.ds(hbase, HALF)], src_v)
            pltpu.sync_copy(dst2d.at[pl.ds(hbase, HALF)], dst_v)
            pltpu.async_copy(y_hbm.at[src_v.at[0]], bufs[0], sems[0])

            def body(i, carry):
                for b in range(2):
                    t = 2 * i + b

                    @pl.when(t + 1 < HALF)
                    def _():
                        pltpu.async_copy(y_hbm.at[src_v.at[t + 1]],
                                         bufs[1 - b], sems[1 - b])

                    pltpu.make_async_copy(y_hbm.at[src_v.at[t]],
                                          bufs[b], sems[b]).wait()
                    pltpu.sync_copy(bufs[b], acc.at[dst_v.at[t]], add=True)
                return carry

            lax.fori_loop(0, HALF // 2, body, 0)
        plsc.subcore_barrier()
        pltpu.sync_copy(acc.at[pl.ds(stripe, STRIPE)],
                        out_hbm.at[pl.ds(stripe, STRIPE)])

    @pl.when(cid == 0)
    def _():
        run(y0, out0)

    @pl.when(cid == 1)
    def _():
        run(y1, out1)


_agg_call = pl.kernel(
    _agg_body,
    out_type=[jax.ShapeDtypeStruct((N_PAD, DH), jnp.float32)] * 2,
    mesh=_MESH,
    name="gcn_sc_agg",
    scratch_types=[
        pltpu.VMEM((HALF, DH), jnp.int32),
        pltpu.VMEM((HALF, DH), jnp.int32),
        pltpu.VMEM((DH, DH), jnp.float32),
        pltpu.VMEM((DH, DH), jnp.float32),
        pltpu.SemaphoreType.DMA,
        pltpu.SemaphoreType.DMA,
        pltpu.VMEM_SHARED((N_PAD, DH), jnp.float32),
    ],
)


def _mm_body(x_ref, w_ref, o_ref):
    o_ref[...] = jnp.dot(x_ref[...], w_ref[...],
                         preferred_element_type=jnp.float32)


_mm_call = pl.pallas_call(
    _mm_body,
    grid=(GRID_M,),
    in_specs=[
        pl.BlockSpec((BM, D), lambda i: (i, 0)),
        pl.BlockSpec((D, D), lambda i: (0, 0)),
    ],
    out_specs=pl.BlockSpec((BM, D), lambda i: (i, 0)),
    out_shape=jax.ShapeDtypeStruct((N, D), jnp.float32),
)


def _scale_body(xw_ref, deg_ref, y0_ref, y1_ref, dinv_ref):
    dinv1 = lax.rsqrt(deg_ref[...][:, 0:1] + 1.0)
    y = xw_ref[...] * dinv1
    y0_ref[...] = y[:, :DH]
    y1_ref[...] = y[:, DH:]
    dinv_ref[...] = jnp.broadcast_to(dinv1, (BM, 16))


_scale_call = pl.pallas_call(
    _scale_body,
    grid=(GRID_M,),
    in_specs=[
        pl.BlockSpec((BM, D), lambda i: (i, 0)),
        pl.BlockSpec((BM, DH), lambda i: (i, 0)),
    ],
    out_specs=[
        pl.BlockSpec((BM, DH), lambda i: (i, 0)),
        pl.BlockSpec((BM, DH), lambda i: (i, 0)),
        pl.BlockSpec((BM, 16), lambda i: (i, 0)),
    ],
    out_shape=[
        jax.ShapeDtypeStruct((N, DH), jnp.float32),
        jax.ShapeDtypeStruct((N, DH), jnp.float32),
        jax.ShapeDtypeStruct((N, 16), jnp.float32),
    ],
)


def _layer2_body(a0_ref, a1_ref, y0_ref, y1_ref, dinv_ref, b1_ref, w2_ref,
                 o0_ref, o1_ref):
    dinv1 = dinv_ref[...][:, 0:1]
    z = jnp.concatenate([a0_ref[...] + y0_ref[...],
                         a1_ref[...] + y1_ref[...]], axis=1)
    h = jnp.maximum(z * dinv1 + b1_ref[...], 0.0)
    y2 = jnp.dot(h, w2_ref[...], preferred_element_type=jnp.float32) * dinv1
    o0_ref[...] = y2[:, :DH]
    o1_ref[...] = y2[:, DH:]


_layer2_call = pl.pallas_call(
    _layer2_body,
    grid=(GRID_M,),
    in_specs=[
        pl.BlockSpec((BM, DH), lambda i: (i, 0)),
        pl.BlockSpec((BM, DH), lambda i: (i, 0)),
        pl.BlockSpec((BM, DH), lambda i: (i, 0)),
        pl.BlockSpec((BM, DH), lambda i: (i, 0)),
        pl.BlockSpec((BM, 16), lambda i: (i, 0)),
        pl.BlockSpec((1, D), lambda i: (0, 0)),
        pl.BlockSpec((D, D), lambda i: (0, 0)),
    ],
    out_specs=[
        pl.BlockSpec((BM, DH), lambda i: (i, 0)),
        pl.BlockSpec((BM, DH), lambda i: (i, 0)),
    ],
    out_shape=[
        jax.ShapeDtypeStruct((N, DH), jnp.float32),
        jax.ShapeDtypeStruct((N, DH), jnp.float32),
    ],
)


def _final_body(a0_ref, a1_ref, y0_ref, y1_ref, dinv_ref, b2_ref, o_ref):
    dinv1 = dinv_ref[...][:, 0:1]
    z = jnp.concatenate([a0_ref[...] + y0_ref[...],
                         a1_ref[...] + y1_ref[...]], axis=1)
    o_ref[...] = z * dinv1 + b2_ref[...]


_final_call = pl.pallas_call(
    _final_body,
    grid=(GRID_M,),
    in_specs=[
        pl.BlockSpec((BM, DH), lambda i: (i, 0)),
        pl.BlockSpec((BM, DH), lambda i: (i, 0)),
        pl.BlockSpec((BM, DH), lambda i: (i, 0)),
        pl.BlockSpec((BM, DH), lambda i: (i, 0)),
        pl.BlockSpec((BM, 16), lambda i: (i, 0)),
        pl.BlockSpec((1, D), lambda i: (0, 0)),
    ],
    out_specs=pl.BlockSpec((BM, D), lambda i: (i, 0)),
    out_shape=jax.ShapeDtypeStruct((N, D), jnp.float32),
)


@jax.jit
def kernel(x, edge_index, W1, b1, W2, b2):
    src = edge_index[0].astype(jnp.int32)
    dst = edge_index[1].astype(jnp.int32)
    pad = ROWS_PAD * DH - E
    src2d = jnp.concatenate([src, jnp.zeros((pad,), jnp.int32)]).reshape(
        ROWS_PAD, DH)
    dst2d = jnp.concatenate([dst, jnp.full((pad,), N, jnp.int32)]).reshape(
        ROWS_PAD, DH)
    zeros = jnp.zeros((STRIPE, DH), jnp.float32)

    ones = jnp.ones((N, DH), jnp.float32)
    deg128, _ = _agg_call(ones, ones, src2d, dst2d, zeros)
    xw1 = _mm_call(x, W1)
    y0, y1, dinv16 = _scale_call(xw1, deg128)
    a0, a1 = _agg_call(y0, y1, src2d, dst2d, zeros)
    z0, z1 = _layer2_call(a0, a1, y0, y1, dinv16,
                          b1.reshape(1, D), W2)
    c0, c1 = _agg_call(z0, z1, src2d, dst2d, zeros)
    return _final_call(c0, c1, z0, z1, dinv16, b2.reshape(1, D))


# speedup vs baseline: 8.1888x; 1.4972x over previous
import jax
import jax.numpy as jnp
from jax import lax
from jax.experimental import pallas as pl
from jax.experimental.pallas import tpu as pltpu
from jax.experimental.pallas import tpu_sc as plsc

N = 10000
E = 160000
D = 256
DH = 128
NS = 16
N_PAD = 10240
STRIPE = N_PAD // NS
ROWS_PAD = 1280
TRIPS = ROWS_PAD // NS
HALF = TRIPS // 2
BM = 512
GRID_M = -(-N // BM)

_MESH = plsc.VectorSubcoreMesh(core_axis_name="c", subcore_axis_name="s")


def _agg_body(y0, y1, src2d, dst2d, zeros, mode, out0, out1,
              src_v, dst_v, rows_a, rows_b, mode_v, sem_a, sem_b,
              acc):
    cid = lax.axis_index("c")
    sid = lax.axis_index("s")
    stripe = pl.multiple_of(sid * STRIPE, 8)
    rbase = pl.multiple_of(sid * TRIPS, 8)

    pltpu.sync_copy(mode, mode_v)
    pltpu.sync_copy(zeros, acc.at[pl.ds(stripe, STRIPE)])
    plsc.subcore_barrier()
    is_deg = mode_v[...][0] == 1

    def run(y_hbm):
        bufs = (rows_a, rows_b)
        sems = (sem_a, sem_b)
        for h in range(2):
            hbase = pl.multiple_of(rbase + h * HALF, 8)
            pltpu.sync_copy(src2d.at[pl.ds(hbase, HALF)], src_v)
            pltpu.sync_copy(dst2d.at[pl.ds(hbase, HALF)], dst_v)
            pltpu.async_copy(y_hbm.at[src_v.at[0]], bufs[0], sems[0])

            def body(i, carry):
                for b in range(2):
                    t = 2 * i + b

                    @pl.when(t + 1 < HALF)
                    def _():
                        pltpu.async_copy(y_hbm.at[src_v.at[t + 1]],
                                         bufs[1 - b], sems[1 - b])

                    pltpu.make_async_copy(y_hbm.at[src_v.at[t]],
                                          bufs[b], sems[b]).wait()
                    pltpu.sync_copy(bufs[b], acc.at[dst_v.at[t]], add=True)
                return carry

            lax.fori_loop(0, HALF // 2, body, 0)

    @pl.when(is_deg)
    def _():
        wid = sid * 2 + cid
        wbase = pl.multiple_of(wid * (ROWS_PAD // 32), 8)
        pltpu.sync_copy(y0.at[pl.ds(0, DH)], rows_a)
        pltpu.sync_copy(dst2d.at[pl.ds(wbase, ROWS_PAD // 32)], dst_v)

        def body(i, carry):
            pltpu.sync_copy(rows_a, acc.at[dst_v.at[i]], add=True)
            return carry

        lax.fori_loop(0, ROWS_PAD // 32, body, 0)

    @pl.when(jnp.logical_and(jnp.logical_not(is_deg), cid == 0))
    def _():
        run(y0)

    @pl.when(jnp.logical_and(jnp.logical_not(is_deg), cid == 1))
    def _():
        run(y1)

    plsc.subcore_barrier()

    @pl.when(cid == 0)
    def _():
        pltpu.sync_copy(acc.at[pl.ds(stripe, STRIPE)],
                        out0.at[pl.ds(stripe, STRIPE)])

    @pl.when(cid == 1)
    def _():
        pltpu.sync_copy(acc.at[pl.ds(stripe, STRIPE)],
                        out1.at[pl.ds(stripe, STRIPE)])


_agg_call = pl.kernel(
    _agg_body,
    out_type=[jax.ShapeDtypeStruct((N_PAD, DH), jnp.float32)] * 2,
    mesh=_MESH,
    name="gcn_sc_agg",
    scratch_types=[
        pltpu.VMEM((HALF, DH), jnp.int32),
        pltpu.VMEM((HALF, DH), jnp.int32),
        pltpu.VMEM((DH, DH), jnp.float32),
        pltpu.VMEM((DH, DH), jnp.float32),
        pltpu.VMEM((16,), jnp.int32),
        pltpu.SemaphoreType.DMA,
        pltpu.SemaphoreType.DMA,
        pltpu.VMEM_SHARED((N_PAD, DH), jnp.float32),
    ],
)


def _mm_body(x_ref, w_ref, o_ref):
    o_ref[...] = jnp.dot(x_ref[...], w_ref[...],
                         preferred_element_type=jnp.float32)


_mm_call = pl.pallas_call(
    _mm_body,
    grid=(GRID_M,),
    in_specs=[
        pl.BlockSpec((BM, D), lambda i: (i, 0)),
        pl.BlockSpec((D, D), lambda i: (0, 0)),
    ],
    out_specs=pl.BlockSpec((BM, D), lambda i: (i, 0)),
    out_shape=jax.ShapeDtypeStruct((N, D), jnp.float32),
)


def _scale_body(xw_ref, dg0_ref, dg1_ref, y0_ref, y1_ref, dinv_ref):
    deg = dg0_ref[...][:, 0:1] + dg1_ref[...][:, 0:1]
    dinv1 = lax.rsqrt(deg + 1.0)
    y = xw_ref[...] * dinv1
    y0_ref[...] = y[:, :DH]
    y1_ref[...] = y[:, DH:]
    dinv_ref[...] = jnp.broadcast_to(dinv1, (BM, 16))


_scale_call = pl.pallas_call(
    _scale_body,
    grid=(GRID_M,),
    in_specs=[
        pl.BlockSpec((BM, D), lambda i: (i, 0)),
        pl.BlockSpec((BM, DH), lambda i: (i, 0)),
        pl.BlockSpec((BM, DH), lambda i: (i, 0)),
    ],
    out_specs=[
        pl.BlockSpec((BM, DH), lambda i: (i, 0)),
        pl.BlockSpec((BM, DH), lambda i: (i, 0)),
        pl.BlockSpec((BM, 16), lambda i: (i, 0)),
    ],
    out_shape=[
        jax.ShapeDtypeStruct((N, DH), jnp.float32),
        jax.ShapeDtypeStruct((N, DH), jnp.float32),
        jax.ShapeDtypeStruct((N, 16), jnp.float32),
    ],
)


def _layer2_body(a0_ref, a1_ref, y0_ref, y1_ref, dinv_ref, b1_ref, w2_ref,
                 o0_ref, o1_ref):
    dinv1 = dinv_ref[...][:, 0:1]
    z = jnp.concatenate([a0_ref[...] + y0_ref[...],
                         a1_ref[...] + y1_ref[...]], axis=1)
    h = jnp.maximum(z * dinv1 + b1_ref[...], 0.0)
    y2 = jnp.dot(h, w2_ref[...], preferred_element_type=jnp.float32) * dinv1
    o0_ref[...] = y2[:, :DH]
    o1_ref[...] = y2[:, DH:]


_layer2_call = pl.pallas_call(
    _layer2_body,
    grid=(GRID_M,),
    in_specs=[
        pl.BlockSpec((BM, DH), lambda i: (i, 0)),
        pl.BlockSpec((BM, DH), lambda i: (i, 0)),
        pl.BlockSpec((BM, DH), lambda i: (i, 0)),
        pl.BlockSpec((BM, DH), lambda i: (i, 0)),
        pl.BlockSpec((BM, 16), lambda i: (i, 0)),
        pl.BlockSpec((1, D), lambda i: (0, 0)),
        pl.BlockSpec((D, D), lambda i: (0, 0)),
    ],
    out_specs=[
        pl.BlockSpec((BM, DH), lambda i: (i, 0)),
        pl.BlockSpec((BM, DH), lambda i: (i, 0)),
    ],
    out_shape=[
        jax.ShapeDtypeStruct((N, DH), jnp.float32),
        jax.ShapeDtypeStruct((N, DH), jnp.float32),
    ],
)


def _final_body(a0_ref, a1_ref, y0_ref, y1_ref, dinv_ref, b2_ref, o_ref):
    dinv1 = dinv_ref[...][:, 0:1]
    z = jnp.concatenate([a0_ref[...] + y0_ref[...],
                         a1_ref[...] + y1_ref[...]], axis=1)
    o_ref[...] = z * dinv1 + b2_ref[...]


_final_call = pl.pallas_call(
    _final_body,
    grid=(GRID_M,),
    in_specs=[
        pl.BlockSpec((BM, DH), lambda i: (i, 0)),
        pl.BlockSpec((BM, DH), lambda i: (i, 0)),
        pl.BlockSpec((BM, DH), lambda i: (i, 0)),
        pl.BlockSpec((BM, DH), lambda i: (i, 0)),
        pl.BlockSpec((BM, 16), lambda i: (i, 0)),
        pl.BlockSpec((1, D), lambda i: (0, 0)),
    ],
    out_specs=pl.BlockSpec((BM, D), lambda i: (i, 0)),
    out_shape=jax.ShapeDtypeStruct((N, D), jnp.float32),
)


@jax.jit
def kernel(x, edge_index, W1, b1, W2, b2):
    src = edge_index[0].astype(jnp.int32)
    dst = edge_index[1].astype(jnp.int32)
    pad = ROWS_PAD * DH - E
    src2d = jnp.concatenate([src, jnp.zeros((pad,), jnp.int32)]).reshape(
        ROWS_PAD, DH)
    dst2d = jnp.concatenate([dst, jnp.full((pad,), N, jnp.int32)]).reshape(
        ROWS_PAD, DH)
    zeros = jnp.zeros((STRIPE, DH), jnp.float32)
    dummy = jnp.ones((N, DH), jnp.float32)
    mode_deg = jnp.ones((16,), jnp.int32)
    mode_agg = jnp.zeros((16,), jnp.int32)

    dg0, dg1 = _agg_call(dummy, dummy, src2d, dst2d, zeros,
                         mode_deg)
    xw1 = _mm_call(x, W1)
    y0, y1, dinv16 = _scale_call(xw1, dg0, dg1)
    a0, a1 = _agg_call(y0, y1, src2d, dst2d, zeros,
                       mode_agg)
    z0, z1 = _layer2_call(a0, a1, y0, y1, dinv16,
                          b1.reshape(1, D), W2)
    c0, c1 = _agg_call(z0, z1, src2d, dst2d, zeros,
                       mode_agg)
    return _final_call(c0, c1, z0, z1, dinv16, b2.reshape(1, D))
